# baseline (speedup 1.0000x reference)
import jax
import jax.numpy as jnp
from jax.experimental import pallas as pl


def _copy_body(fm_ref, lm_ref, fo_ref, lo_ref):
    fo_ref[...] = fm_ref[...]
    lo_ref[...] = lm_ref[...]


def kernel(features, labels, features_memory, labels_memory):
    batch = features.shape[0]
    dim = features_memory.shape[1]
    blk = 2048
    grid = (batch // blk,)
    feats_out, labels_out = pl.pallas_call(
        _copy_body,
        grid=grid,
        out_shape=(
            jax.ShapeDtypeStruct((batch, dim), features_memory.dtype),
            jax.ShapeDtypeStruct((batch, 1), labels_memory.dtype),
        ),
        in_specs=[
            pl.BlockSpec((blk, dim), lambda i: (i, 0)),
            pl.BlockSpec((blk, 1), lambda i: (i, 0)),
        ],
        out_specs=(
            pl.BlockSpec((blk, dim), lambda i: (i, 0)),
            pl.BlockSpec((blk, 1), lambda i: (i, 0)),
        ),
    )(features_memory, labels_memory)
    return feats_out, labels_out
